# Initial kernel scaffold; baseline (speedup 1.0000x reference)
import jax
import jax.numpy as jnp
from jax.experimental import pallas as pl


def kernel(x):
    raise NotImplementedError("write your pallas kernel here")


# speedup vs baseline: 5.3291x; 5.3291x over previous
import jax
import jax.numpy as jnp
from jax import lax
from jax.experimental import pallas as pl
from jax.experimental.pallas import tpu as pltpu

_MASK_PERCENT = 0.5
_CB = 128


def _body(r_ref, x_ref, e_ref, o_ref):
    freq = r_ref.shape[0]
    keep_thresh = int(_MASK_PERCENT * freq)
    r = r_ref[...]
    a = r[:, None, :]
    b = r[None, :, :]
    ii = lax.broadcasted_iota(jnp.int32, (freq, freq, r.shape[1]), 0)
    jj = lax.broadcasted_iota(jnp.int32, (freq, freq, r.shape[1]), 1)
    less = (b < a) | ((b == a) & (jj < ii))
    rank = jnp.sum(less.astype(jnp.float32), axis=1)
    m = (rank >= keep_thresh).astype(jnp.float32)
    me = jax.lax.dot(m, e_ref[...],
                     preferred_element_type=jnp.float32)
    o_ref[...] = x_ref[...] * me


def kernel(x):
    freq, batch, n1, n2, length = x.shape
    ncols = batch * n1 * n2
    rkey = jax.random.key(42)
    r = jax.random.uniform(rkey, (freq, batch, n1, n2), dtype=jnp.float32)
    r2 = r.reshape(freq, ncols)
    x2 = x.reshape(freq, ncols * length)
    e = (jnp.arange(_CB, dtype=jnp.int32)[:, None]
         == (jnp.arange(_CB * length, dtype=jnp.int32)[None, :] // length)
         ).astype(jnp.float32)
    grid = ncols // _CB
    out = pl.pallas_call(
        _body,
        grid=(grid,),
        in_specs=[
            pl.BlockSpec((freq, _CB), lambda j: (0, j)),
            pl.BlockSpec((freq, _CB * length), lambda j: (0, j)),
            pl.BlockSpec((_CB, _CB * length), lambda j: (0, 0)),
        ],
        out_specs=pl.BlockSpec((freq, _CB * length), lambda j: (0, j)),
        out_shape=jax.ShapeDtypeStruct((freq, ncols * length), jnp.float32),
    )(r2, x2, e)
    return out.reshape(x.shape)
